# baseline (speedup 1.0000x reference)
import functools

import jax
import jax.numpy as jnp
from jax import lax
from jax.experimental import pallas as pl
from jax.experimental.pallas import tpu as pltpu
from jax.experimental.pallas import tpu_sc as plsc

CUT_NUM = 20
HIDDEN = 1024
EMBED_SCALE = 0.1
B = 1024
S = 256

NC = 2
NS = 16
NW = NC * NS
B_PER_W = B // NW

BB = 4


def _make_sc_gather():
    mesh = plsc.VectorSubcoreMesh(core_axis_name="c", subcore_axis_name="s")

    @functools.partial(
        pl.kernel,
        mesh=mesh,
        out_type=jax.ShapeDtypeStruct((2 * B, HIDDEN), jnp.float32),
        scratch_types=[
            pltpu.VMEM((2 * B_PER_W,), jnp.int32),
            pltpu.VMEM((2 * B_PER_W, HIDDEN), jnp.float32),
            pltpu.SemaphoreType.DMA,
        ],
    )
    def sc_gather(idx_hbm, cat_table_hbm, emb_out, idx, rows, sem):
        wid = lax.axis_index("s") * NC + lax.axis_index("c")
        nrows = 2 * B_PER_W
        base = wid * nrows
        pltpu.sync_copy(idx_hbm.at[pl.ds(base, nrows)], idx)
        pltpu.async_copy(cat_table_hbm.at[idx], rows, sem).wait()
        pltpu.sync_copy(rows, emb_out.at[pl.ds(base, nrows)])

    return sc_gather


_sc_gather_cache = []


def _get_sc_gather():
    if not _sc_gather_cache:
        _sc_gather_cache.append(_make_sc_gather())
    return _sc_gather_cache[0]


def _add_body(hid_ref, h_ref, w_ref, out_ref):
    emb = (h_ref[0, :, :, :] + w_ref[0, :, :, :]) * EMBED_SCALE
    out_ref[...] = hid_ref[...] + emb


_broadcast_add = pl.pallas_call(
    _add_body,
    grid=(B // BB,),
    in_specs=[
        pl.BlockSpec((BB, S, HIDDEN), lambda i: (i, 0, 0)),
        pl.BlockSpec((1, BB, 1, HIDDEN), lambda i: (0, i, 0, 0)),
        pl.BlockSpec((1, BB, 1, HIDDEN), lambda i: (1, i, 0, 0)),
    ],
    out_specs=pl.BlockSpec((BB, S, HIDDEN), lambda i: (i, 0, 0)),
    out_shape=jax.ShapeDtypeStruct((B, S, HIDDEN), jnp.float32),
)


def kernel(hidden_states, patch_positions, h_table, w_table):
    pp = patch_positions.astype(jnp.int32)
    idx_all = jnp.concatenate([pp[:, 0], pp[:, 1] + CUT_NUM])
    cat_table = jnp.concatenate([h_table, w_table], axis=0)
    emb = _get_sc_gather()(idx_all, cat_table)
    emb4 = emb.reshape(2, B, 1, HIDDEN)
    return _broadcast_add(hidden_states, emb4, emb4)


# speedup vs baseline: 1.0409x; 1.0409x over previous
import functools

import jax
import jax.numpy as jnp
from jax import lax
from jax.experimental import pallas as pl
from jax.experimental.pallas import tpu as pltpu
from jax.experimental.pallas import tpu_sc as plsc

CUT_NUM = 20
HIDDEN = 1024
EMBED_SCALE = 0.1
B = 1024
S = 256

NC = 2
NS = 16
NW = NC * NS
B_PER_W = B // NW

BB = 8


def _make_sc_gather():
    mesh = plsc.VectorSubcoreMesh(core_axis_name="c", subcore_axis_name="s")

    @functools.partial(
        pl.kernel,
        mesh=mesh,
        out_type=jax.ShapeDtypeStruct((2 * B, HIDDEN), jnp.float32),
        scratch_types=[
            pltpu.VMEM((2 * B_PER_W,), jnp.int32),
            pltpu.VMEM((2 * B_PER_W, HIDDEN), jnp.float32),
            pltpu.SemaphoreType.DMA,
            pltpu.SemaphoreType.DMA,
        ],
    )
    def sc_gather(idx_hbm, cat_table_hbm, emb_out, idx, rows, sem_g, sem_c):
        wid = lax.axis_index("s") * NC + lax.axis_index("c")
        nrows = 2 * B_PER_W
        half = nrows // 2
        base = wid * nrows
        pltpu.sync_copy(idx_hbm.at[pl.ds(base, nrows)], idx)
        pltpu.async_copy(cat_table_hbm.at[idx.at[pl.ds(0, half)]],
                         rows.at[pl.ds(0, half)], sem_g).wait()
        g1 = pltpu.async_copy(cat_table_hbm.at[idx.at[pl.ds(half, half)]],
                              rows.at[pl.ds(half, half)], sem_g)
        c0 = pltpu.async_copy(rows.at[pl.ds(0, half)],
                              emb_out.at[pl.ds(base, half)], sem_c)
        g1.wait()
        c1 = pltpu.async_copy(rows.at[pl.ds(half, half)],
                              emb_out.at[pl.ds(base + half, half)], sem_c)
        c0.wait()
        c1.wait()

    return sc_gather


_sc_gather_cache = []


def _get_sc_gather():
    if not _sc_gather_cache:
        _sc_gather_cache.append(_make_sc_gather())
    return _sc_gather_cache[0]


def _add_body(hid_ref, emb_ref, out_ref):
    i = pl.program_id(0)
    h = emb_ref[pl.ds(i * BB, BB), :]
    w = emb_ref[pl.ds(B + i * BB, BB), :]
    emb = (h + w) * EMBED_SCALE
    out_ref[...] = hid_ref[...] + emb[:, None, :]


_broadcast_add = pl.pallas_call(
    _add_body,
    grid=(B // BB,),
    in_specs=[
        pl.BlockSpec((BB, S, HIDDEN), lambda i: (i, 0, 0)),
        pl.BlockSpec((2 * B, HIDDEN), lambda i: (0, 0)),
    ],
    out_specs=pl.BlockSpec((BB, S, HIDDEN), lambda i: (i, 0, 0)),
    out_shape=jax.ShapeDtypeStruct((B, S, HIDDEN), jnp.float32),
)


def kernel(hidden_states, patch_positions, h_table, w_table):
    pp = patch_positions.astype(jnp.int32)
    idx_all = jnp.concatenate([pp[:, 0], pp[:, 1] + CUT_NUM])
    cat_table = jnp.concatenate([h_table, w_table], axis=0)
    emb = _get_sc_gather()(idx_all, cat_table)
    return _broadcast_add(hidden_states, emb)


# speedup vs baseline: 1.0422x; 1.0013x over previous
import functools

import jax
import jax.numpy as jnp
from jax import lax
from jax.experimental import pallas as pl
from jax.experimental.pallas import tpu as pltpu
from jax.experimental.pallas import tpu_sc as plsc

CUT_NUM = 20
HIDDEN = 1024
EMBED_SCALE = 0.1
B = 1024
S = 256

NC = 2
NS = 16
NW = NC * NS
B_PER_W = B // NW

BB = 8


def _make_sc_gather():
    mesh = plsc.VectorSubcoreMesh(core_axis_name="c", subcore_axis_name="s")

    @functools.partial(
        pl.kernel,
        mesh=mesh,
        out_type=jax.ShapeDtypeStruct((2 * B, HIDDEN), jnp.float32),
        scratch_types=[
            pltpu.VMEM((2 * B_PER_W,), jnp.int32),
            pltpu.VMEM((2 * B_PER_W, HIDDEN), jnp.float32),
            pltpu.SemaphoreType.DMA,
        ],
    )
    def sc_gather(idx_hbm, cat_table_hbm, emb_out, idx, rows, sem):
        wid = lax.axis_index("s") * NC + lax.axis_index("c")
        nrows = 2 * B_PER_W
        base = wid * nrows
        pltpu.sync_copy(idx_hbm.at[pl.ds(base, nrows)], idx)
        pltpu.async_copy(cat_table_hbm.at[idx], rows, sem).wait()
        pltpu.sync_copy(rows, emb_out.at[pl.ds(base, nrows)])

    return sc_gather


_sc_gather_cache = []


def _get_sc_gather():
    if not _sc_gather_cache:
        _sc_gather_cache.append(_make_sc_gather())
    return _sc_gather_cache[0]


def _add_body(hid_ref, emb_ref, out_ref):
    i = pl.program_id(0)
    h = emb_ref[pl.ds(i * BB, BB), :]
    w = emb_ref[pl.ds(B + i * BB, BB), :]
    emb = (h + w) * EMBED_SCALE
    out_ref[...] = hid_ref[...] + emb[:, None, :]


_broadcast_add = pl.pallas_call(
    _add_body,
    grid=(B // BB,),
    in_specs=[
        pl.BlockSpec((BB, S, HIDDEN), lambda i: (i, 0, 0)),
        pl.BlockSpec((2 * B, HIDDEN), lambda i: (0, 0)),
    ],
    out_specs=pl.BlockSpec((BB, S, HIDDEN), lambda i: (i, 0, 0)),
    out_shape=jax.ShapeDtypeStruct((B, S, HIDDEN), jnp.float32),
)


def kernel(hidden_states, patch_positions, h_table, w_table):
    pp = patch_positions.astype(jnp.int32)
    idx_all = jnp.concatenate([pp[:, 0], pp[:, 1] + CUT_NUM])
    cat_table = jnp.concatenate([h_table, w_table], axis=0)
    emb = _get_sc_gather()(idx_all, cat_table)
    return _broadcast_add(hidden_states, emb)
